# Initial kernel scaffold; baseline (speedup 1.0000x reference)
import jax
import jax.numpy as jnp
from jax.experimental import pallas as pl


def kernel(node_h, edge_index, edge_attr, batch, fallback_num_graphs, params):
    raise NotImplementedError("write your pallas kernel here")


# speedup vs baseline: 2.8728x; 2.8728x over previous
import functools

import jax
import jax.numpy as jnp
from jax import lax
from jax.experimental import pallas as pl
from jax.experimental.pallas import tpu as pltpu
from jax.experimental.pallas import tpu_sc as plsc

H = 128
EH = 16
E = 320000
N = 10000
G = 2500
EDGE_SCALE = 0.1

NC = 2
NS = 16
NW = NC * NS
EPW = E // NW
CH = 80
NCH = EPW // CH
RPT = 632
RPT_LAST = N - (NS - 1) * RPT

EB = 2560
EGRID = E // EB
NB = 2000
NGRID = N // NB
FB = 400
FGRID = N // FB


def _gelu(x):
    return 0.5 * x * (1.0 + lax.erf(x * 0.7071067811865476))


def _ln(x, g, b, eps=1e-5):
    mu = jnp.mean(x, axis=-1, keepdims=True)
    var = jnp.var(x, axis=-1, keepdims=True)
    return (x - mu) / jnp.sqrt(var + eps) * g + b


def _full(x):
    return pl.BlockSpec(x.shape, lambda *_: (0,) * x.ndim)


def _edge_in_body(ea_ref, w_ref, b_ref, g_ref, bb_ref, out_ref):
    x = ea_ref[...] @ w_ref[...] + b_ref[...]
    out_ref[...] = _ln(_gelu(x), g_ref[...], bb_ref[...])


def _edge_in(edge_attr, W, b, g, bb):
    return pl.pallas_call(
        _edge_in_body,
        grid=(EGRID,),
        in_specs=[pl.BlockSpec((EB, EH), lambda i: (i, 0)),
                  _full(W), _full(b), _full(g), _full(bb)],
        out_specs=pl.BlockSpec((EB, EH), lambda i: (i, 0)),
        out_shape=jax.ShapeDtypeStruct((E, EH), jnp.float32),
    )(edge_attr, W, b, g, bb)


def _sc_gather(h, src2, dst2):
    mesh = plsc.VectorSubcoreMesh(core_axis_name="c", subcore_axis_name="s",
                                  num_cores=NC, num_subcores=NS)

    @functools.partial(
        pl.kernel,
        out_type=(jax.ShapeDtypeStruct((E, H), jnp.float32),
                  jax.ShapeDtypeStruct((E, H), jnp.float32)),
        mesh=mesh,
        scratch_types=[
            pltpu.VMEM((EPW,), jnp.int32),
            pltpu.VMEM((EPW,), jnp.int32),
            pltpu.VMEM((CH, H), jnp.float32),
            pltpu.VMEM((CH, H), jnp.float32),
            pltpu.SemaphoreType.DMA,
            pltpu.SemaphoreType.DMA,
        ],
    )
    def k(h_hbm, src_hbm, dst_hbm, hs_out, hd_out, sidx, didx, srows, drows,
          sem_s, sem_d):
        wid = lax.axis_index("s") * NC + lax.axis_index("c")
        base = wid * EPW
        pltpu.sync_copy(src_hbm.at[wid], sidx)
        pltpu.sync_copy(dst_hbm.at[wid], didx)

        def body(j, carry):
            off = j * CH
            cs = pltpu.async_copy(h_hbm.at[sidx.at[pl.ds(off, CH)]], srows,
                                  sem_s)
            cd = pltpu.async_copy(h_hbm.at[didx.at[pl.ds(off, CH)]], drows,
                                  sem_d)
            cs.wait()
            pltpu.sync_copy(srows, hs_out.at[pl.ds(base + off, CH)])
            cd.wait()
            pltpu.sync_copy(drows, hd_out.at[pl.ds(base + off, CH)])
            return carry

        lax.fori_loop(0, NCH, body, 0)

    return k(h, src2, dst2)


def _sc_scatter(m, dst3, zeros_nh):
    mesh = plsc.VectorSubcoreMesh(core_axis_name="c", subcore_axis_name="s",
                                  num_cores=NC, num_subcores=NS)

    @functools.partial(
        pl.kernel,
        out_type=jax.ShapeDtypeStruct((2 * N, H), jnp.float32),
        mesh=mesh,
        scratch_types=[
            pltpu.VMEM((NCH, CH), jnp.int32),
            pltpu.VMEM((CH, H), jnp.float32),
            pltpu.VMEM_SHARED((N, H), jnp.float32),
        ],
    )
    def k(m_hbm, dst_hbm, zeros_hbm, out_hbm, idxs, rows, shared):
        cid = lax.axis_index("c")
        sid = lax.axis_index("s")
        wid = sid * NC + cid
        base = wid * EPW
        row0 = sid * RPT

        @pl.when(sid < NS - 1)
        def _():
            pltpu.sync_copy(zeros_hbm.at[pl.ds(row0, RPT)],
                            shared.at[pl.ds(row0, RPT)])

        @pl.when(sid == NS - 1)
        def _():
            pltpu.sync_copy(zeros_hbm.at[pl.ds((NS - 1) * RPT, RPT_LAST)],
                            shared.at[pl.ds((NS - 1) * RPT, RPT_LAST)])

        pltpu.sync_copy(dst_hbm.at[wid], idxs)
        plsc.subcore_barrier()

        def body(j, carry):
            pltpu.sync_copy(m_hbm.at[pl.ds(base + j * CH, CH)], rows)
            pltpu.sync_copy(rows, shared.at[idxs.at[j]], add=True)
            return carry

        lax.fori_loop(0, NCH, body, 0)
        plsc.subcore_barrier()

        obase = cid * N + row0

        @pl.when(sid < NS - 1)
        def _():
            pltpu.sync_copy(shared.at[pl.ds(row0, RPT)],
                            out_hbm.at[pl.ds(obase, RPT)])

        @pl.when(sid == NS - 1)
        def _():
            pltpu.sync_copy(shared.at[pl.ds((NS - 1) * RPT, RPT_LAST)],
                            out_hbm.at[pl.ds(cid * N + (NS - 1) * RPT,
                                             RPT_LAST)])

    return k(m, dst3, zeros_nh)


def _edge_layer_body(hs_ref, hd_ref, e_ref, wqs, wqd, wqe, bq, emw2, emb2,
                     eng, enb, w1s, w1e, b1, w2, b2, e_out, m_out):
    hs = hs_ref[...]
    hd = hd_ref[...]
    e = e_ref[...]
    q = hs @ wqs[...] + hd @ wqd[...] + e @ wqe[...] + bq[...]
    gate = jax.nn.sigmoid(q[:, EH:EH + 1])
    delta = _gelu(q[:, :EH]) @ emw2[...] + emb2[...]
    e2 = _ln(e + EDGE_SCALE * delta * gate, eng[...], enb[...])
    t = _gelu(hs @ w1s[...] + e2 @ w1e[...] + b1[...])
    m_out[...] = t @ w2[...] + b2[...]
    e_out[...] = e2


def _edge_layer(hs, hd, e, weights):
    wspecs = [_full(w) for w in weights]
    return pl.pallas_call(
        _edge_layer_body,
        grid=(EGRID,),
        in_specs=[pl.BlockSpec((EB, H), lambda i: (i, 0)),
                  pl.BlockSpec((EB, H), lambda i: (i, 0)),
                  pl.BlockSpec((EB, EH), lambda i: (i, 0))] + wspecs,
        out_specs=(pl.BlockSpec((EB, EH), lambda i: (i, 0)),
                   pl.BlockSpec((EB, H), lambda i: (i, 0))),
        out_shape=(jax.ShapeDtypeStruct((E, EH), jnp.float32),
                   jax.ShapeDtypeStruct((E, H), jnp.float32)),
    )(hs, hd, e, *weights)


def _node_update_body(h_ref, a0_ref, a1_ref, w1h, w1a, b1, w2, b2, ng, nb,
                      out_ref):
    h = h_ref[...]
    agg = a0_ref[...] + a1_ref[...]
    u = _gelu(h @ w1h[...] + agg @ w1a[...] + b1[...]) @ w2[...] + b2[...]
    out_ref[...] = _ln(h + u, ng[...], nb[...])


def _node_update(h, a0, a1, weights):
    wspecs = [_full(w) for w in weights]
    return pl.pallas_call(
        _node_update_body,
        grid=(NGRID,),
        in_specs=[pl.BlockSpec((NB, H), lambda i: (i, 0)),
                  pl.BlockSpec((NB, H), lambda i: (i, 0)),
                  pl.BlockSpec((NB, H), lambda i: (i, 0))] + wspecs,
        out_specs=pl.BlockSpec((NB, H), lambda i: (i, 0)),
        out_shape=jax.ShapeDtypeStruct((N, H), jnp.float32),
    )(h, a0, a1, *weights)


def _final_body(h_ref, b_ref, og, ob, h_out, mix_out, summ, cnt):
    i = pl.program_id(0)
    hn = _ln(h_ref[...], og[...], ob[...])
    h_out[...] = hn

    @pl.when(i == 0)
    def _():
        summ[...] = jnp.zeros_like(summ)
        cnt[...] = jnp.zeros_like(cnt)

    bids = b_ref[0, 0, :]
    gid = lax.broadcasted_iota(jnp.int32, (G, FB), 0)
    S = (gid == bids[None, :]).astype(jnp.float32)
    summ[...] += jnp.dot(S, hn)
    cnt[...] += jnp.dot(S, jnp.ones((FB, H), jnp.float32))

    @pl.when(i == FGRID - 1)
    def _():
        mix_out[...] = summ[...] / jnp.clip(cnt[...], 1.0, None)


def _final_pool(h, batch3, og, ob):
    return pl.pallas_call(
        _final_body,
        grid=(FGRID,),
        in_specs=[pl.BlockSpec((FB, H), lambda i: (i, 0)),
                  pl.BlockSpec((1, 1, FB), lambda i: (i, 0, 0)),
                  _full(og), _full(ob)],
        out_specs=(pl.BlockSpec((FB, H), lambda i: (i, 0)),
                   pl.BlockSpec((G, H), lambda i: (0, 0))),
        out_shape=(jax.ShapeDtypeStruct((N, H), jnp.float32),
                   jax.ShapeDtypeStruct((G, H), jnp.float32)),
        scratch_shapes=[pltpu.VMEM((G, H), jnp.float32),
                        pltpu.VMEM((G, H), jnp.float32)],
    )(h, batch3, og, ob)


def _row(x):
    return x.reshape(1, -1).astype(jnp.float32)


def kernel(node_h, edge_index, edge_attr, batch, fallback_num_graphs, params):
    src = edge_index[0].astype(jnp.int32)
    dst = edge_index[1].astype(jnp.int32)
    src2 = src.reshape(NW, EPW)
    dst2 = dst.reshape(NW, EPW)
    dst3 = dst.reshape(NW, NCH, CH)
    batch3 = batch.astype(jnp.int32).reshape(FGRID, 1, FB)
    zeros_nh = jnp.zeros((N, H), jnp.float32)

    e = _edge_in(edge_attr, params['edge_in_W'], _row(params['edge_in_b']),
                 _row(params['edge_norm_g']), _row(params['edge_norm_b']))

    h = node_h
    for lp in params['layers']:
        em_W1 = lp['em_W1']
        eg_W = lp['eg_W']
        wqs = jnp.zeros((H, 32), jnp.float32)
        wqs = wqs.at[:, :EH].set(em_W1[:H]).at[:, EH].set(eg_W[:H, 0])
        wqd = jnp.zeros((H, 32), jnp.float32)
        wqd = wqd.at[:, :EH].set(em_W1[H:2 * H]).at[:, EH].set(eg_W[H:2 * H, 0])
        wqe = jnp.zeros((EH, 32), jnp.float32)
        wqe = wqe.at[:, :EH].set(em_W1[2 * H:]).at[:, EH].set(eg_W[2 * H:, 0])
        bq = jnp.zeros((1, 32), jnp.float32)
        bq = bq.at[0, :EH].set(lp['em_b1']).at[0, EH].set(lp['eg_b'][0])

        edge_w = [wqs, wqd, wqe, bq, lp['em_W2'], _row(lp['em_b2']),
                  _row(lp['enorm_g']), _row(lp['enorm_b']),
                  lp['msg_W1'][:H], lp['msg_W1'][H:], _row(lp['msg_b1']),
                  lp['msg_W2'], _row(lp['msg_b2'])]
        upd_w = [lp['upd_W1'][:H], lp['upd_W1'][H:], _row(lp['upd_b1']),
                 lp['upd_W2'], _row(lp['upd_b2']),
                 _row(lp['norm_g']), _row(lp['norm_b'])]

        hs, hd = _sc_gather(h, src2, dst2)
        e, m = _edge_layer(hs, hd, e, edge_w)
        parts = _sc_scatter(m, dst3, zeros_nh)
        h = _node_update(h, parts[:N], parts[N:], upd_w)

    h_out, mix = _final_pool(h, batch3, _row(params['out_norm_g']),
                             _row(params['out_norm_b']))
    scale = fallback_num_graphs.astype(jnp.float32) / jnp.float32(G) \
        if hasattr(fallback_num_graphs, 'astype') \
        else jnp.float32(fallback_num_graphs) / jnp.float32(G)
    mix = mix * scale
    return h_out, mix
